# baseline (speedup 1.0000x reference)
import functools

import jax
import jax.numpy as jnp
from jax import lax
from jax.experimental import pallas as pl
from jax.experimental.pallas import tpu as pltpu
from jax.experimental.pallas import tpu_sc as plsc

K = 8
L = 16
NUM_CORES = 2
NUM_SUBCORES = 16
NW = NUM_CORES * NUM_SUBCORES


_SORT8 = ((0, 1), (2, 3), (4, 5), (6, 7),
          (0, 2), (1, 3), (4, 6), (5, 7),
          (1, 2), (5, 6),
          (0, 4), (1, 5), (2, 6), (3, 7),
          (2, 4), (3, 5),
          (1, 2), (3, 4), (5, 6))
_BITONIC8 = ((0, 4), (1, 5), (2, 6), (3, 7),
             (0, 2), (1, 3), (4, 6), (5, 7),
             (0, 1), (2, 3), (4, 5), (6, 7))


def _apply_net(net, v):
    v = list(v)
    for a, b in net:
        lo = jnp.minimum(v[a], v[b])
        hi = jnp.maximum(v[a], v[b])
        v[a], v[b] = lo, hi
    return v


def _merge_bottom8(S, C):
    t = [jnp.minimum(S[i], C[K - 1 - i]) for i in range(K)]
    return _apply_net(_BITONIC8, t)


def _bcast_lane(v, j):
    idx = jnp.full((L,), j, dtype=jnp.int32)
    return v.at[idx].get(mode="promise_in_bounds")


def _sc_kernel(num_b, num_i, inp_hbm, w_hbm, out_hbm,
               inp_v, w_v, out_v, pw_v, mw_v, pi_v, mi_v):
    wid = lax.axis_index("s") * NUM_CORES + lax.axis_index("c")
    chunk_w = num_i * L
    chunk_o = num_b * L
    pltpu.sync_copy(inp_hbm, inp_v)
    pltpu.sync_copy(w_hbm.at[pl.ds(wid * chunk_w, chunk_w)], w_v)

    inf = jnp.full((L,), jnp.inf, dtype=jnp.float32)

    def w_body(i, carry):
        w = w_v[pl.ds(i * L, L)]
        pw = jnp.maximum(w, 0.0)
        pw_v[pl.ds(i * L, L)] = pw
        mw_v[pl.ds(i * L, L)] = pw - w
        return carry

    lax.fori_loop(0, num_i, w_body, 0)

    def b_body(b, carry):
        base = b * num_i

        def x_body(k, carry2):
            xv = inp_v[pl.ds(base + k * L, L)]
            pi_v[pl.ds(k * L, L)] = jnp.maximum(xv + 4.0, 0.0)
            mi_v[pl.ds(k * L, L)] = jnp.maximum(4.0 - xv, 0.0)
            return carry2

        lax.fori_loop(0, num_i // L, x_body, 0)

        def blk_body(k, S):
            piv = pi_v[pl.ds(k * L, L)]
            miv = mi_v[pl.ds(k * L, L)]
            for q in range(L // 4):
                h = q % 2
                Sp = list(S[h * K:(h + 1) * K])
                Sm = list(S[(2 + h) * K:(3 + h) * K])
                Cp, Cm = [], []
                for j in range(4 * q, 4 * q + 4):
                    pi = _bcast_lane(piv, j)
                    mi = _bcast_lane(miv, j)
                    pw = pw_v[pl.ds((k * L + j) * L, L)]
                    mw = mw_v[pl.ds((k * L + j) * L, L)]
                    Cp += [pi + pw, mi + mw]
                    Cm += [pi + mw, mi + pw]
                Sp = _merge_bottom8(Sp, _apply_net(_SORT8, Cp))
                Sm = _merge_bottom8(Sm, _apply_net(_SORT8, Cm))
                S = (tuple(S[:h * K]) + tuple(Sp) + tuple(S[(h + 1) * K:(2 + h) * K])
                     + tuple(Sm) + tuple(S[(3 + h) * K:]))
            return S

        S = lax.fori_loop(0, num_i // L, blk_body, (inf,) * (4 * K))
        SP = _merge_bottom8(list(S[:K]), list(S[K:2 * K]))
        SM = _merge_bottom8(list(S[2 * K:3 * K]), list(S[3 * K:]))
        resP = SP[0]
        for j in range(1, K):
            resP = resP + SP[j]
        resM = SM[0]
        for j in range(1, K):
            resM = resM + SM[j]
        out_v[pl.ds(b * L, L)] = (resP - resM) * (1.0 / K)
        return carry

    lax.fori_loop(0, num_b, b_body, 0)
    pltpu.sync_copy(out_v, out_hbm.at[pl.ds(wid * chunk_o, chunk_o)])


def kernel(inputp, weight):
    num_b, num_i = inputp.shape
    _, num_o = weight.shape
    w_chunks = weight.reshape(num_i, NW, L).transpose(1, 0, 2).reshape(-1)
    inp_flat = inputp.reshape(-1)
    mesh = plsc.VectorSubcoreMesh(
        core_axis_name="c", subcore_axis_name="s",
        num_cores=NUM_CORES, num_subcores=NUM_SUBCORES)
    f = pl.kernel(
        functools.partial(_sc_kernel, num_b, num_i),
        out_type=jax.ShapeDtypeStruct((NW * num_b * L,), jnp.float32),
        mesh=mesh,
        scratch_types=[
            pltpu.VMEM((num_b * num_i,), jnp.float32),
            pltpu.VMEM((num_i * L,), jnp.float32),
            pltpu.VMEM((num_b * L,), jnp.float32),
            pltpu.VMEM((num_i * L,), jnp.float32),
            pltpu.VMEM((num_i * L,), jnp.float32),
            pltpu.VMEM((num_i,), jnp.float32),
            pltpu.VMEM((num_i,), jnp.float32),
        ],
    )
    out = f(inp_flat, w_chunks)
    return out.reshape(NW, num_b, L).transpose(1, 0, 2).reshape(num_b, num_o)


# speedup vs baseline: 1.1059x; 1.1059x over previous
import functools

import jax
import jax.numpy as jnp
from jax import lax
from jax.experimental import pallas as pl
from jax.experimental.pallas import tpu as pltpu
from jax.experimental.pallas import tpu_sc as plsc

K = 8
L = 16
NUM_CORES = 2
NUM_SUBCORES = 16
NW = NUM_CORES * NUM_SUBCORES


_SORT8 = ((0, 1), (2, 3), (4, 5), (6, 7),
          (0, 2), (1, 3), (4, 6), (5, 7),
          (1, 2), (5, 6),
          (0, 4), (1, 5), (2, 6), (3, 7),
          (2, 4), (3, 5),
          (1, 2), (3, 4), (5, 6))
_BITONIC8 = ((0, 4), (1, 5), (2, 6), (3, 7),
             (0, 2), (1, 3), (4, 6), (5, 7),
             (0, 1), (2, 3), (4, 5), (6, 7))


def _apply_net(net, v):
    v = list(v)
    for a, b in net:
        lo = jnp.minimum(v[a], v[b])
        hi = jnp.maximum(v[a], v[b])
        v[a], v[b] = lo, hi
    return v


def _merge_bottom8(S, C):
    t = [jnp.minimum(S[i], C[K - 1 - i]) for i in range(K)]
    return _apply_net(_BITONIC8, t)


def _bcast_lane(v, j):
    idx = jnp.full((L,), j, dtype=jnp.int32)
    return v.at[idx].get(mode="promise_in_bounds")


def _sc_kernel(num_b, num_i, inp_hbm, w_hbm, out_hbm,
               inp_v, w_v, out_v, pw_v, mw_v, pi_v, mi_v):
    wid = lax.axis_index("s") * NUM_CORES + lax.axis_index("c")
    chunk_w = num_i * L
    chunk_o = num_b * L
    pltpu.sync_copy(inp_hbm, inp_v)
    pltpu.sync_copy(w_hbm.at[pl.ds(wid * chunk_w, chunk_w)], w_v)

    inf = jnp.full((L,), jnp.inf, dtype=jnp.float32)

    def w_body(i, carry):
        w = w_v[pl.ds(i * L, L)]
        pw = jnp.maximum(w, 0.0)
        pw_v[pl.ds(i * L, L)] = pw
        mw_v[pl.ds(i * L, L)] = pw - w
        return carry

    lax.fori_loop(0, num_i, w_body, 0)

    def b_body(b, carry):
        base = b * num_i

        def x_body(k, carry2):
            xv = inp_v[pl.ds(base + k * L, L)]
            pi_v[pl.ds(k * L, L)] = jnp.maximum(xv + 4.0, 0.0)
            mi_v[pl.ds(k * L, L)] = jnp.maximum(4.0 - xv, 0.0)
            return carry2

        lax.fori_loop(0, num_i // L, x_body, 0)

        def blk_body(k, S):
            piv = pi_v[pl.ds(k * L, L)]
            miv = mi_v[pl.ds(k * L, L)]
            for q in range(L // 4):
                Sp, Sm = list(S[:K]), list(S[K:])
                Cp, Cm = [], []
                for j in range(4 * q, 4 * q + 4):
                    pi = _bcast_lane(piv, j)
                    mi = _bcast_lane(miv, j)
                    pw = pw_v[pl.ds((k * L + j) * L, L)]
                    mw = mw_v[pl.ds((k * L + j) * L, L)]
                    Cp += [pi + pw, mi + mw]
                    Cm += [pi + mw, mi + pw]
                Sp = _merge_bottom8(Sp, _apply_net(_SORT8, Cp))
                Sm = _merge_bottom8(Sm, _apply_net(_SORT8, Cm))
                S = tuple(Sp) + tuple(Sm)
            return S

        S = lax.fori_loop(0, num_i // L, blk_body, (inf,) * (2 * K))
        SP, SM = S[:K], S[K:]
        resP = SP[0]
        for j in range(1, K):
            resP = resP + SP[j]
        resM = SM[0]
        for j in range(1, K):
            resM = resM + SM[j]
        out_v[pl.ds(b * L, L)] = (resP - resM) * (1.0 / K)
        return carry

    lax.fori_loop(0, num_b, b_body, 0)
    pltpu.sync_copy(out_v, out_hbm.at[pl.ds(wid * chunk_o, chunk_o)])


def kernel(inputp, weight):
    num_b, num_i = inputp.shape
    _, num_o = weight.shape
    w_chunks = weight.reshape(num_i, NW, L).transpose(1, 0, 2).reshape(-1)
    inp_flat = inputp.reshape(-1)
    mesh = plsc.VectorSubcoreMesh(
        core_axis_name="c", subcore_axis_name="s",
        num_cores=NUM_CORES, num_subcores=NUM_SUBCORES)
    f = pl.kernel(
        functools.partial(_sc_kernel, num_b, num_i),
        out_type=jax.ShapeDtypeStruct((NW * num_b * L,), jnp.float32),
        mesh=mesh,
        scratch_types=[
            pltpu.VMEM((num_b * num_i,), jnp.float32),
            pltpu.VMEM((num_i * L,), jnp.float32),
            pltpu.VMEM((num_b * L,), jnp.float32),
            pltpu.VMEM((num_i * L,), jnp.float32),
            pltpu.VMEM((num_i * L,), jnp.float32),
            pltpu.VMEM((num_i,), jnp.float32),
            pltpu.VMEM((num_i,), jnp.float32),
        ],
    )
    out = f(inp_flat, w_chunks)
    return out.reshape(NW, num_b, L).transpose(1, 0, 2).reshape(num_b, num_o)


# speedup vs baseline: 1.2841x; 1.1611x over previous
import functools

import jax
import jax.numpy as jnp
from jax import lax
from jax.experimental import pallas as pl
from jax.experimental.pallas import tpu as pltpu
from jax.experimental.pallas import tpu_sc as plsc

K = 8
L = 16
NUM_CORES = 2
NUM_SUBCORES = 16
NW = NUM_CORES * NUM_SUBCORES


_SORT8 = ((0, 1), (2, 3), (4, 5), (6, 7),
          (0, 2), (1, 3), (4, 6), (5, 7),
          (1, 2), (5, 6),
          (0, 4), (1, 5), (2, 6), (3, 7),
          (2, 4), (3, 5),
          (1, 2), (3, 4), (5, 6))
_BITONIC8 = ((0, 4), (1, 5), (2, 6), (3, 7),
             (0, 2), (1, 3), (4, 6), (5, 7),
             (0, 1), (2, 3), (4, 5), (6, 7))


def _apply_net(net, v):
    v = list(v)
    for a, b in net:
        lo = jnp.minimum(v[a], v[b])
        hi = jnp.maximum(v[a], v[b])
        v[a], v[b] = lo, hi
    return v


def _merge_bottom8(S, C):
    t = [jnp.minimum(S[i], C[K - 1 - i]) for i in range(K)]
    return _apply_net(_BITONIC8, t)


def _bcast_lane(v, j):
    idx = jnp.full((L,), j, dtype=jnp.int32)
    return v.at[idx].get(mode="promise_in_bounds")


def _sc_kernel(num_b, num_i, inp_hbm, w_hbm, out_hbm,
               inp_v, w_v, out_v, pw_v, mw_v, pi_v, mi_v):
    wid = lax.axis_index("s") * NUM_CORES + lax.axis_index("c")
    chunk_w = num_i * L
    chunk_o = num_b * L
    pltpu.sync_copy(inp_hbm, inp_v)
    pltpu.sync_copy(w_hbm.at[pl.ds(wid * chunk_w, chunk_w)], w_v)

    inf = jnp.full((L,), jnp.inf, dtype=jnp.float32)

    def w_body(i, carry):
        w = w_v[pl.ds(i * L, L)]
        pw = jnp.maximum(w, 0.0)
        pw_v[pl.ds(i * L, L)] = pw
        mw_v[pl.ds(i * L, L)] = pw - w
        return carry

    lax.fori_loop(0, num_i, w_body, 0)

    def b_body(b, carry):
        base = b * num_i

        def x_body(k, carry2):
            xv = inp_v[pl.ds(base + k * L, L)]
            pi_v[pl.ds(k * L, L)] = jnp.maximum(xv + 4.0, 0.0)
            mi_v[pl.ds(k * L, L)] = jnp.maximum(4.0 - xv, 0.0)
            return carry2

        lax.fori_loop(0, num_i // L, x_body, 0)

        def blk_body(k, S):
            piv = pi_v[pl.ds(k * L, L)]
            miv = mi_v[pl.ds(k * L, L)]
            Tp, Tm = [], []
            for q in range(L // 4):
                Cp, Cm = [], []
                for j in range(4 * q, 4 * q + 4):
                    pi = _bcast_lane(piv, j)
                    mi = _bcast_lane(miv, j)
                    pw = pw_v[pl.ds((k * L + j) * L, L)]
                    mw = mw_v[pl.ds((k * L + j) * L, L)]
                    Cp += [pi + pw, mi + mw]
                    Cm += [pi + mw, mi + pw]
                Tp.append(_apply_net(_SORT8, Cp))
                Tm.append(_apply_net(_SORT8, Cm))
            Tp1 = _merge_bottom8(Tp[0], Tp[1])
            Tp2 = _merge_bottom8(Tp[2], Tp[3])
            Tm1 = _merge_bottom8(Tm[0], Tm[1])
            Tm2 = _merge_bottom8(Tm[2], Tm[3])
            Sp = _merge_bottom8(list(S[:K]), _merge_bottom8(Tp1, Tp2))
            Sm = _merge_bottom8(list(S[K:]), _merge_bottom8(Tm1, Tm2))
            return tuple(Sp) + tuple(Sm)

        S = lax.fori_loop(0, num_i // L, blk_body, (inf,) * (2 * K))
        SP, SM = S[:K], S[K:]
        resP = SP[0]
        for j in range(1, K):
            resP = resP + SP[j]
        resM = SM[0]
        for j in range(1, K):
            resM = resM + SM[j]
        out_v[pl.ds(b * L, L)] = (resP - resM) * (1.0 / K)
        return carry

    lax.fori_loop(0, num_b, b_body, 0)
    pltpu.sync_copy(out_v, out_hbm.at[pl.ds(wid * chunk_o, chunk_o)])


def kernel(inputp, weight):
    num_b, num_i = inputp.shape
    _, num_o = weight.shape
    w_chunks = weight.reshape(num_i, NW, L).transpose(1, 0, 2).reshape(-1)
    inp_flat = inputp.reshape(-1)
    mesh = plsc.VectorSubcoreMesh(
        core_axis_name="c", subcore_axis_name="s",
        num_cores=NUM_CORES, num_subcores=NUM_SUBCORES)
    f = pl.kernel(
        functools.partial(_sc_kernel, num_b, num_i),
        out_type=jax.ShapeDtypeStruct((NW * num_b * L,), jnp.float32),
        mesh=mesh,
        scratch_types=[
            pltpu.VMEM((num_b * num_i,), jnp.float32),
            pltpu.VMEM((num_i * L,), jnp.float32),
            pltpu.VMEM((num_b * L,), jnp.float32),
            pltpu.VMEM((num_i * L,), jnp.float32),
            pltpu.VMEM((num_i * L,), jnp.float32),
            pltpu.VMEM((num_i,), jnp.float32),
            pltpu.VMEM((num_i,), jnp.float32),
        ],
    )
    out = f(inp_flat, w_chunks)
    return out.reshape(NW, num_b, L).transpose(1, 0, 2).reshape(num_b, num_o)
